# Initial kernel scaffold; baseline (speedup 1.0000x reference)
import functools

import jax
import jax.numpy as jnp
from jax import lax
from jax.experimental import pallas as pl
from jax.experimental.pallas import tpu as pltpu
from jax.experimental.pallas import tpu_sc as plsc

N = 10000
E = 320000
D_IN = 128
D_DEG = 64
HID = 256
D_OUT = 128

NPAD = 10240
RB = 256
NBLK = NPAD // RB
NC = 2
NS = 16
NTILES = NC * NS
CHUNK = 128
CPT = 79
EPT = CPT * CHUNK
EPAD = NTILES * EPT
ROWS_PER_TILE = NPAD // NS


def _deg_body(dst_hbm, out_hbm, dstbuf, hist, sem):
    del sem
    c = lax.axis_index("c")
    s = lax.axis_index("s")
    w = s * NC + c
    pltpu.sync_copy(dst_hbm.at[w], dstbuf)

    def zero_body(i, carry):
        hist[pl.ds(i * 16, 16)] = jnp.zeros((16,), jnp.float32)
        return carry

    lax.fori_loop(0, NPAD // 16, zero_body, 0)
    ones = jnp.ones((16,), jnp.float32)

    def chunk_body(j, carry):
        for k in range(CHUNK // 16):
            idx = dstbuf[j, pl.ds(k * 16, 16)]
            plsc.addupdate_scatter(hist, [idx], ones)
        return carry

    lax.fori_loop(0, CPT, chunk_body, 0)
    pltpu.sync_copy(hist, out_hbm.at[w])


_deg_kernel = functools.partial(
    pl.kernel,
    out_type=jax.ShapeDtypeStruct((NTILES, NPAD), jnp.float32),
    mesh=plsc.VectorSubcoreMesh(core_axis_name="c", subcore_axis_name="s"),
    scratch_types=[
        pltpu.VMEM((CPT, CHUNK), jnp.int32),
        pltpu.VMEM((NPAD,), jnp.float32),
        pltpu.SemaphoreType.DMA,
    ],
)(_deg_body)


def _make_spmm(f2):
    def body(hp_hbm, src_hbm, dst_hbm, out_hbm, srcbuf, dstbuf, rowbuf, acc,
             sem):
        c = lax.axis_index("c")
        s = lax.axis_index("s")
        r0 = s * ROWS_PER_TILE
        pltpu.sync_copy(hp_hbm.at[pl.ds(c * NPAD + r0, ROWS_PER_TILE)],
                        acc.at[pl.ds(r0, ROWS_PER_TILE)])
        plsc.subcore_barrier()

        def do_slab(slab):
            pltpu.sync_copy(src_hbm.at[c, slab], srcbuf)
            pltpu.sync_copy(dst_hbm.at[slab], dstbuf)

            def chunk_body(j, carry):
                pltpu.async_copy(hp_hbm.at[srcbuf.at[j]], rowbuf, sem).wait()
                pltpu.sync_copy(rowbuf, acc.at[dstbuf.at[j]], add=True)
                return carry

            lax.fori_loop(0, CPT, chunk_body, 0)

        do_slab(s)
        do_slab(s + NS)
        plsc.subcore_barrier()
        pltpu.sync_copy(acc.at[pl.ds(r0, ROWS_PER_TILE)],
                        out_hbm.at[pl.ds(c * NPAD + r0, ROWS_PER_TILE)])

    return functools.partial(
        pl.kernel,
        out_type=jax.ShapeDtypeStruct((NC * NPAD, f2), jnp.float32),
        mesh=plsc.VectorSubcoreMesh(core_axis_name="c", subcore_axis_name="s"),
        scratch_types=[
            pltpu.VMEM((CPT, CHUNK), jnp.int32),
            pltpu.VMEM((CPT, CHUNK), jnp.int32),
            pltpu.VMEM((CHUNK, f2), jnp.float32),
            pltpu.VMEM_SHARED((NPAD, f2), jnp.float32),
            pltpu.SemaphoreType.DMA,
        ],
    )(body)


_spmm_l1 = _make_spmm(HID // 2)
_spmm_l2 = _make_spmm(D_OUT // 2)


def _dinv_from_parts(degp):
    return lax.rsqrt(jnp.sum(degp, axis=0) + 1.0)


def _mm1_body(x_ref, ds_ref, degp_ref, w1a_ref, w1b_ref, out_ref):
    dinv = _dinv_from_parts(degp_ref[...])
    ds = ds_ref[0, 0, :]
    oh = (ds[:, None] == lax.broadcasted_iota(jnp.int32, (1, D_DEG), 1)
          ).astype(jnp.float32)
    h = (jnp.dot(x_ref[...], w1a_ref[...], precision=lax.Precision.HIGHEST,
                 preferred_element_type=jnp.float32)
         + jnp.dot(oh, w1b_ref[...], precision=lax.Precision.HIGHEST,
                   preferred_element_type=jnp.float32))
    hp = dinv[:, None] * h
    out_ref[0] = hp[:, :HID // 2]
    out_ref[1] = hp[:, HID // 2:]


def _mm1(x_pad, ds_pad, degp, w1a, w1b):
    return pl.pallas_call(
        _mm1_body,
        grid=(NBLK,),
        in_specs=[
            pl.BlockSpec((RB, D_IN), lambda i: (i, 0)),
            pl.BlockSpec((1, 1, RB), lambda i: (i, 0, 0)),
            pl.BlockSpec((NTILES, RB), lambda i: (0, i)),
            pl.BlockSpec((D_IN, HID), lambda i: (0, 0)),
            pl.BlockSpec((D_DEG, HID), lambda i: (0, 0)),
        ],
        out_specs=pl.BlockSpec((NC, RB, HID // 2), lambda i: (0, i, 0)),
        out_shape=jax.ShapeDtypeStruct((NC, NPAD, HID // 2), jnp.float32),
    )(x_pad, ds_pad, degp, w1a, w1b)


def _mm2_body(acc_ref, degp_ref, w2_ref, b1_ref, out_ref):
    dinv = _dinv_from_parts(degp_ref[...])
    a = jnp.concatenate([acc_ref[0], acc_ref[1]], axis=1)
    m = jnp.maximum(dinv[:, None] * a + b1_ref[...], 0.0)
    h2 = jnp.dot(m, w2_ref[...], precision=lax.Precision.HIGHEST,
                 preferred_element_type=jnp.float32)
    hp = dinv[:, None] * h2
    out_ref[0] = hp[:, :D_OUT // 2]
    out_ref[1] = hp[:, D_OUT // 2:]


def _mm2(acc1, degp, w2, b1):
    return pl.pallas_call(
        _mm2_body,
        grid=(NBLK,),
        in_specs=[
            pl.BlockSpec((NC, RB, HID // 2), lambda i: (0, i, 0)),
            pl.BlockSpec((NTILES, RB), lambda i: (0, i)),
            pl.BlockSpec((HID, D_OUT), lambda i: (0, 0)),
            pl.BlockSpec((1, HID), lambda i: (0, 0)),
        ],
        out_specs=pl.BlockSpec((NC, RB, D_OUT // 2), lambda i: (0, i, 0)),
        out_shape=jax.ShapeDtypeStruct((NC, NPAD, D_OUT // 2), jnp.float32),
    )(acc1, degp, w2, b1)


def _mm3_body(acc_ref, degp_ref, b2_ref, out_ref):
    dinv = _dinv_from_parts(degp_ref[...])
    a = jnp.concatenate([acc_ref[0], acc_ref[1]], axis=1)
    out_ref[...] = dinv[:, None] * a + b2_ref[...]


def _mm3(acc2, degp, b2):
    return pl.pallas_call(
        _mm3_body,
        grid=(NBLK,),
        in_specs=[
            pl.BlockSpec((NC, RB, D_OUT // 2), lambda i: (0, i, 0)),
            pl.BlockSpec((NTILES, RB), lambda i: (0, i)),
            pl.BlockSpec((1, D_OUT), lambda i: (0, 0)),
        ],
        out_specs=pl.BlockSpec((RB, D_OUT), lambda i: (i, 0)),
        out_shape=jax.ShapeDtypeStruct((NPAD, D_OUT), jnp.float32),
    )(acc2, degp, b2)


def kernel(x, edge_index, degree_set, W1, b1, W2, b2):
    x = x.astype(jnp.float32)
    src = edge_index[0].astype(jnp.int32)
    dst = edge_index[1].astype(jnp.int32)
    degree_set = degree_set.astype(jnp.int32)

    pad = EPAD - E
    srcp = jnp.concatenate([src, jnp.full((pad,), N, jnp.int32)])
    dstp = jnp.concatenate([dst, jnp.full((pad,), N, jnp.int32)])
    src_r = srcp.reshape(NTILES, CPT, CHUNK)
    dst_r = dstp.reshape(NTILES, CPT, CHUNK)
    src2 = jnp.stack([src_r, src_r + NPAD])

    x_pad = jnp.pad(x, ((0, NPAD - N), (0, 0)))
    ds_pad = jnp.pad(degree_set, (0, NPAD - N)).reshape(NBLK, 1, RB)

    degp = _deg_kernel(dst_r)

    hp1 = _mm1(x_pad, ds_pad, degp, W1[:D_IN], W1[D_IN:])
    acc1 = _spmm_l1(hp1.reshape(NC * NPAD, HID // 2), src2, dst_r)
    hp2 = _mm2(acc1.reshape(NC, NPAD, HID // 2), degp, W2,
               b1.reshape(1, HID))
    acc2 = _spmm_l2(hp2.reshape(NC * NPAD, D_OUT // 2), src2, dst_r)
    out = _mm3(acc2.reshape(NC, NPAD, D_OUT // 2), degp,
               b2.reshape(1, D_OUT))
    return out[:N]


# speedup vs baseline: 10.9694x; 10.9694x over previous
import functools

import jax
import jax.numpy as jnp
from jax import lax
from jax.experimental import pallas as pl
from jax.experimental.pallas import tpu as pltpu
from jax.experimental.pallas import tpu_sc as plsc

N = 10000
E = 320000
D_IN = 128
D_DEG = 64
HID = 256
D_OUT = 128

NPAD = 10240
RB = 256
NBLK = NPAD // RB
NC = 2
NS = 16
NTILES = NC * NS
CHUNK = 128
CPT = 79
EPT = CPT * CHUNK
EPAD = NTILES * EPT
ROWS_PER_TILE = NPAD // NS


def _deg_body(dst_hbm, out_hbm, dstbuf, hist, sem):
    del sem
    c = lax.axis_index("c")
    s = lax.axis_index("s")
    w = s * NC + c
    pltpu.sync_copy(dst_hbm.at[w], dstbuf)

    def zero_body(i, carry):
        hist[pl.ds(i * 16, 16)] = jnp.zeros((16,), jnp.float32)
        return carry

    lax.fori_loop(0, NPAD // 16, zero_body, 0)
    ones = jnp.ones((16,), jnp.float32)

    def chunk_body(j, carry):
        for k in range(CHUNK // 16):
            idx = dstbuf[j, pl.ds(k * 16, 16)]
            plsc.addupdate_scatter(hist, [idx], ones)
        return carry

    lax.fori_loop(0, CPT, chunk_body, 0)
    pltpu.sync_copy(hist, out_hbm.at[w])


_deg_kernel = functools.partial(
    pl.kernel,
    out_type=jax.ShapeDtypeStruct((NTILES, NPAD), jnp.float32),
    mesh=plsc.VectorSubcoreMesh(core_axis_name="c", subcore_axis_name="s"),
    scratch_types=[
        pltpu.VMEM((CPT, CHUNK), jnp.int32),
        pltpu.VMEM((NPAD,), jnp.float32),
        pltpu.SemaphoreType.DMA,
    ],
    compiler_params=pltpu.CompilerParams(needs_layout_passes=False),
)(_deg_body)


def _spmm_l1_body(hp_hbm, src_hbm, dst_hbm, out_hbm, srcbuf, dstbuf, rowbuf,
                  acc, sem):
    c = lax.axis_index("c")
    s = lax.axis_index("s")
    r0 = s * ROWS_PER_TILE
    pltpu.sync_copy(hp_hbm.at[pl.ds(c * NPAD + r0, ROWS_PER_TILE)],
                    acc.at[pl.ds(r0, ROWS_PER_TILE)])
    plsc.subcore_barrier()

    def do_slab(slab):
        pltpu.sync_copy(src_hbm.at[c, slab], srcbuf)
        pltpu.sync_copy(dst_hbm.at[slab], dstbuf)

        def chunk_body(j, carry):
            pltpu.async_copy(hp_hbm.at[srcbuf.at[j]], rowbuf, sem).wait()
            pltpu.sync_copy(rowbuf, acc.at[dstbuf.at[j]], add=True)
            return carry

        lax.fori_loop(0, CPT, chunk_body, 0)

    do_slab(s)
    do_slab(s + NS)
    plsc.subcore_barrier()
    pltpu.sync_copy(acc.at[pl.ds(r0, ROWS_PER_TILE)],
                    out_hbm.at[pl.ds(c * NPAD + r0, ROWS_PER_TILE)])


_spmm_l1 = functools.partial(
    pl.kernel,
    out_type=jax.ShapeDtypeStruct((NC * NPAD, HID // 2), jnp.float32),
    mesh=plsc.VectorSubcoreMesh(core_axis_name="c", subcore_axis_name="s"),
    scratch_types=[
        pltpu.VMEM((CPT, CHUNK), jnp.int32),
        pltpu.VMEM((CPT, CHUNK), jnp.int32),
        pltpu.VMEM((CHUNK, HID // 2), jnp.float32),
        pltpu.VMEM_SHARED((NPAD, HID // 2), jnp.float32),
        pltpu.SemaphoreType.DMA,
    ],
    compiler_params=pltpu.CompilerParams(needs_layout_passes=False),
)(_spmm_l1_body)


def _spmm_l2_body(hp_hbm, src_hbm, dst_hbm, out_hbm, srcbuf, dstbuf, rowbuf,
                  acc, sem):
    c = lax.axis_index("c")
    s = lax.axis_index("s")
    r0 = s * ROWS_PER_TILE
    pltpu.sync_copy(hp_hbm.at[pl.ds(r0, ROWS_PER_TILE)],
                    acc.at[pl.ds(r0, ROWS_PER_TILE)])
    plsc.subcore_barrier()
    slab = c * NS + s
    pltpu.sync_copy(src_hbm.at[slab], srcbuf)
    pltpu.sync_copy(dst_hbm.at[slab], dstbuf)

    def chunk_body(j, carry):
        pltpu.async_copy(hp_hbm.at[srcbuf.at[j]], rowbuf, sem).wait()
        pltpu.sync_copy(rowbuf, acc.at[dstbuf.at[j]], add=True)
        return carry

    lax.fori_loop(0, CPT, chunk_body, 0)
    plsc.subcore_barrier()
    pltpu.sync_copy(acc.at[pl.ds(r0, ROWS_PER_TILE)],
                    out_hbm.at[c, pl.ds(r0, ROWS_PER_TILE)])


_spmm_l2 = functools.partial(
    pl.kernel,
    out_type=jax.ShapeDtypeStruct((NC, NPAD, D_OUT), jnp.float32),
    mesh=plsc.VectorSubcoreMesh(core_axis_name="c", subcore_axis_name="s"),
    scratch_types=[
        pltpu.VMEM((CPT, CHUNK), jnp.int32),
        pltpu.VMEM((CPT, CHUNK), jnp.int32),
        pltpu.VMEM((CHUNK, D_OUT), jnp.float32),
        pltpu.VMEM_SHARED((NPAD, D_OUT), jnp.float32),
        pltpu.SemaphoreType.DMA,
    ],
    compiler_params=pltpu.CompilerParams(needs_layout_passes=False),
)(_spmm_l2_body)


def _dinv_from_parts(degp):
    return lax.rsqrt(jnp.sum(degp, axis=0) + 1.0)


def _mm1_body(x_ref, ds_ref, degp_ref, w1a_ref, w1b_ref, out_ref):
    dinv = _dinv_from_parts(degp_ref[...])
    ds = ds_ref[0, 0, :]
    oh = (ds[:, None] == lax.broadcasted_iota(jnp.int32, (1, D_DEG), 1)
          ).astype(jnp.float32)
    h = (jnp.dot(x_ref[...], w1a_ref[...], precision=lax.Precision.HIGHEST,
                 preferred_element_type=jnp.float32)
         + jnp.dot(oh, w1b_ref[...], precision=lax.Precision.HIGHEST,
                   preferred_element_type=jnp.float32))
    hp = dinv[:, None] * h
    out_ref[0] = hp[:, :HID // 2]
    out_ref[1] = hp[:, HID // 2:]


def _mm1(x_pad, ds_pad, degp, w1a, w1b):
    return pl.pallas_call(
        _mm1_body,
        grid=(NBLK,),
        in_specs=[
            pl.BlockSpec((RB, D_IN), lambda i: (i, 0)),
            pl.BlockSpec((1, 1, RB), lambda i: (i, 0, 0)),
            pl.BlockSpec((NTILES, RB), lambda i: (0, i)),
            pl.BlockSpec((D_IN, HID), lambda i: (0, 0)),
            pl.BlockSpec((D_DEG, HID), lambda i: (0, 0)),
        ],
        out_specs=pl.BlockSpec((NC, RB, HID // 2), lambda i: (0, i, 0)),
        out_shape=jax.ShapeDtypeStruct((NC, NPAD, HID // 2), jnp.float32),
    )(x_pad, ds_pad, degp, w1a, w1b)


def _mm2_body(acc_ref, degp_ref, w2_ref, b1_ref, out_ref):
    dinv = _dinv_from_parts(degp_ref[...])
    a = jnp.concatenate([acc_ref[0], acc_ref[1]], axis=1)
    m = jnp.maximum(dinv[:, None] * a + b1_ref[...], 0.0)
    h2 = jnp.dot(m, w2_ref[...], precision=lax.Precision.HIGHEST,
                 preferred_element_type=jnp.float32)
    out_ref[...] = dinv[:, None] * h2


def _mm2(acc1, degp, w2, b1):
    return pl.pallas_call(
        _mm2_body,
        grid=(NBLK,),
        in_specs=[
            pl.BlockSpec((NC, RB, HID // 2), lambda i: (0, i, 0)),
            pl.BlockSpec((NTILES, RB), lambda i: (0, i)),
            pl.BlockSpec((HID, D_OUT), lambda i: (0, 0)),
            pl.BlockSpec((1, HID), lambda i: (0, 0)),
        ],
        out_specs=pl.BlockSpec((RB, D_OUT), lambda i: (i, 0)),
        out_shape=jax.ShapeDtypeStruct((NPAD, D_OUT), jnp.float32),
    )(acc1, degp, w2, b1)


def _mm3_body(acc_ref, hp2_ref, degp_ref, b2_ref, out_ref):
    dinv = _dinv_from_parts(degp_ref[...])
    a = acc_ref[0] + acc_ref[1] - hp2_ref[...]
    out_ref[...] = dinv[:, None] * a + b2_ref[...]


def _mm3(acc2, hp2, degp, b2):
    return pl.pallas_call(
        _mm3_body,
        grid=(NBLK,),
        in_specs=[
            pl.BlockSpec((NC, RB, D_OUT), lambda i: (0, i, 0)),
            pl.BlockSpec((RB, D_OUT), lambda i: (i, 0)),
            pl.BlockSpec((NTILES, RB), lambda i: (0, i)),
            pl.BlockSpec((1, D_OUT), lambda i: (0, 0)),
        ],
        out_specs=pl.BlockSpec((RB, D_OUT), lambda i: (i, 0)),
        out_shape=jax.ShapeDtypeStruct((NPAD, D_OUT), jnp.float32),
    )(acc2, hp2, degp, b2)


def kernel(x, edge_index, degree_set, W1, b1, W2, b2):
    x = x.astype(jnp.float32)
    src = edge_index[0].astype(jnp.int32)
    dst = edge_index[1].astype(jnp.int32)
    degree_set = degree_set.astype(jnp.int32)

    pad = EPAD - E
    srcp = jnp.concatenate([src, jnp.full((pad,), N, jnp.int32)])
    dstp = jnp.concatenate([dst, jnp.full((pad,), N, jnp.int32)])
    src_r = srcp.reshape(NTILES, CPT, CHUNK)
    dst_r = dstp.reshape(NTILES, CPT, CHUNK)
    src2 = jnp.stack([src_r, src_r + NPAD])

    x_pad = jnp.pad(x, ((0, NPAD - N), (0, 0)))
    ds_pad = jnp.pad(degree_set, (0, NPAD - N)).reshape(NBLK, 1, RB)

    degp = _deg_kernel(dst_r)

    hp1 = _mm1(x_pad, ds_pad, degp, W1[:D_IN], W1[D_IN:])
    acc1 = _spmm_l1(hp1.reshape(NC * NPAD, HID // 2), src2, dst_r)
    hp2 = _mm2(acc1.reshape(NC, NPAD, HID // 2), degp, W2,
               b1.reshape(1, HID))
    acc2 = _spmm_l2(hp2, src_r, dst_r)
    out = _mm3(acc2, hp2, degp, b2.reshape(1, D_OUT))
    return out[:N]
